# baseline (speedup 1.0000x reference)
import jax
import jax.numpy as jnp
from jax.experimental import pallas as pl
from jax.experimental.pallas import tpu as pltpu

H = 50
HP = 56
G = 4 * HP


KP = 120


def _sig(x):
    return 0.5 * jnp.tanh(0.5 * x) + 0.5


def _gates(g):
    i = _sig(g[0:HP])
    f = _sig(g[HP:2 * HP])
    u = jnp.tanh(g[2 * HP:3 * HP])
    o = _sig(g[3 * HP:4 * HP])
    return i, f, u, o


def _lstm_fused_kernel(x_ref, wi0_ref, wbig_ref, hh0_ref, c2i_ref,
                       wfc_ref, bfc_ref, out_ref,
                       hh_ref, c1_ref, c2_ref):
    tc = pl.program_id(1)
    num_tc = pl.num_programs(1)
    tt = x_ref.shape[0]
    Bb = x_ref.shape[1]

    def _xdot(xt):
        return jax.lax.dot_general(wi0_ref[...], xt, (((1,), (1,)), ((), ())),
                                   preferred_element_type=jnp.float32)

    @pl.when(tc == 0)
    def _init():
        z = jnp.zeros((HP, Bb), jnp.float32)
        hh_ref[...] = jnp.broadcast_to(hh0_ref[...], (KP, Bb))
        c1_ref[...] = z
        c2_ref[...] = jnp.broadcast_to(c2i_ref[...], (HP, Bb))

    def body(k, carry):
        xg = _xdot(x_ref[k])
        for s in range(2):
            sl = slice(128 * s, 128 * (s + 1))
            hh = hh_ref[:, sl]
            gb = jnp.dot(wbig_ref[...], hh, preferred_element_type=jnp.float32)
            g0 = xg[:, sl] + gb[0:G]
            g1 = gb[G:2 * G]
            i0, f0, u0, o0 = _gates(g0)
            i1, f1, u1, o1 = _gates(g1)
            c1n = f0 * c1_ref[:, sl] + i0 * u0
            c2n = f1 * c2_ref[:, sl] + i1 * u1
            h1n = o0 * jnp.tanh(c1n)
            h2n = o1 * jnp.tanh(c2n)
            c1_ref[:, sl] = c1n
            c2_ref[:, sl] = c2n
            hh_ref[0:HP, sl] = h1n
            hh_ref[HP:2 * HP, sl] = h2n
        return 0

    jax.lax.fori_loop(0, tt, body, 0, unroll=64)

    @pl.when(tc == num_tc - 1)
    def _final():
        gb = jnp.dot(wbig_ref[...], hh_ref[...],
                     preferred_element_type=jnp.float32)
        g1 = gb[G:2 * G]
        i1, f1, u1, o1 = _gates(g1)
        c2n = f1 * c2_ref[...] + i1 * u1
        h2n = o1 * jnp.tanh(c2n)
        out_ref[...] = jnp.dot(wfc_ref[...], h2n,
                               preferred_element_type=jnp.float32) + bfc_ref[...]


def _pad_gate_rows(w):
    k = w.shape[1]
    return jnp.pad(w.reshape(4, H, k), ((0, 0), (0, HP - H), (0, 0))).reshape(G, k)


def kernel(x, w_ih0, w_hh0, b_ih0, b_hh0, w_ih1, w_hh1, b_ih1, b_hh1,
           w_fc, b_fc):
    B, T = x.shape[0], x.shape[1]
    x2 = x.reshape(B, T, -1)
    I = x2.shape[-1]
    xT = jnp.swapaxes(x2, 0, 1)

    NB = 2
    Bb = B // NB
    TT = 64
    TC = T // TT

    wi0 = _pad_gate_rows(w_ih0)
    wh0 = _pad_gate_rows(jnp.pad(w_hh0, ((0, 0), (0, HP - H))))
    b0 = jnp.pad((b_ih0 + b_hh0).reshape(4, H),
                 ((0, 0), (0, HP - H))).reshape(G, 1)
    wi1 = _pad_gate_rows(jnp.pad(w_ih1, ((0, 0), (0, HP - H))))
    wh1 = _pad_gate_rows(jnp.pad(w_hh1, ((0, 0), (0, HP - H))))
    b1 = jnp.pad((b_ih1 + b_hh1).reshape(4, H),
                 ((0, 0), (0, HP - H))).reshape(G, 1)
    b1v = b_ih1 + b_hh1
    c2i = -(jax.nn.sigmoid(b1v[0:H]) * jnp.tanh(b1v[2 * H:3 * H])
            / jax.nn.sigmoid(b1v[H:2 * H]))
    c2i = jnp.pad(c2i, (0, HP - H)).reshape(HP, 1)
    wbig = jnp.concatenate([
        jnp.concatenate([wh0, jnp.zeros_like(wh0)], axis=1),
        jnp.concatenate([wi1, wh1], axis=1),
    ], axis=0)
    bias_col = jnp.concatenate([b0, b1], axis=0)
    wbig = jnp.concatenate(
        [wbig, bias_col,
         jnp.zeros((2 * G, KP - 2 * HP - 1), jnp.float32)], axis=1)
    hh0 = jnp.zeros((KP, 1), jnp.float32).at[2 * HP, 0].set(1.0)
    wfc = jnp.pad(w_fc, ((0, 0), (0, HP - H)))
    O = wfc.shape[0]
    bfc = b_fc.reshape(O, 1)

    full = lambda a: pl.BlockSpec(a.shape, lambda b, t: (0,) * a.ndim)

    outT = pl.pallas_call(
        _lstm_fused_kernel,
        out_shape=jax.ShapeDtypeStruct((O, B), jnp.float32),
        grid=(NB, TC),
        in_specs=[
            pl.BlockSpec((TT, Bb, I), lambda b, t: (t, b, 0)),
            full(wi0), full(wbig), full(hh0), full(c2i),
            full(wfc), full(bfc),
        ],
        out_specs=pl.BlockSpec((O, Bb), lambda b, t: (0, b)),
        scratch_shapes=[
            pltpu.VMEM((KP, Bb), jnp.float32),
            pltpu.VMEM((HP, Bb), jnp.float32),
            pltpu.VMEM((HP, Bb), jnp.float32),
        ],
        compiler_params=pltpu.CompilerParams(
            dimension_semantics=("parallel", "arbitrary"),
            vmem_limit_bytes=56 * 1024 * 1024,
        ),
        name="lstm2_fused",
    )(xT, wi0, wbig, hh0, c2i, wfc, bfc)

    return outT.T.reshape(-1, 16, 9)


# speedup vs baseline: 1.2460x; 1.2460x over previous
import jax
import jax.numpy as jnp
from jax.experimental import pallas as pl
from jax.experimental.pallas import tpu as pltpu

H = 50
HP = 56
G = 4 * HP


KP = 120


def _sig(x):
    return 0.5 * jnp.tanh(0.5 * x) + 0.5


def _gates(g):
    i = _sig(g[0:HP])
    f = _sig(g[HP:2 * HP])
    u = jnp.tanh(g[2 * HP:3 * HP])
    o = _sig(g[3 * HP:4 * HP])
    return i, f, u, o


def _lstm_fused_kernel(x_ref, wi0_ref, wbig_ref, hh0_ref, c2i_ref,
                       wfc_ref, bfc_ref, out_ref,
                       hh_ref, c1_ref, c2_ref):
    tc = pl.program_id(1)
    num_tc = pl.num_programs(1)
    tt = x_ref.shape[0]
    Bb = x_ref.shape[1]

    def _xdot(xt):
        return jax.lax.dot_general(wi0_ref[...], xt, (((1,), (1,)), ((), ())),
                                   preferred_element_type=jnp.float32)

    @pl.when(tc == 0)
    def _init():
        z = jnp.zeros((HP, Bb), jnp.float32)
        hh_ref[...] = jnp.broadcast_to(hh0_ref[...], (KP, Bb))
        c1_ref[...] = z
        c2_ref[...] = jnp.broadcast_to(c2i_ref[...], (HP, Bb))

    def body(k, carry):
        xg = _xdot(x_ref[k])
        for s in range(2):
            sl = slice(128 * s, 128 * (s + 1))
            hh = hh_ref[:, sl]
            gb = jnp.dot(wbig_ref[...], hh, preferred_element_type=jnp.float32)
            g0 = xg[:, sl] + gb[0:G]
            g1 = gb[G:2 * G]
            i0, f0, u0, o0 = _gates(g0)
            i1, f1, u1, o1 = _gates(g1)
            c1n = f0 * c1_ref[:, sl] + i0 * u0
            c2n = f1 * c2_ref[:, sl] + i1 * u1
            h1n = o0 * jnp.tanh(c1n)
            h2n = o1 * jnp.tanh(c2n)
            c1_ref[:, sl] = c1n
            c2_ref[:, sl] = c2n
            hh_ref[0:HP, sl] = h1n
            hh_ref[HP:2 * HP, sl] = h2n
        return 0

    jax.lax.fori_loop(0, tt, body, 0, unroll=64)

    @pl.when(tc == num_tc - 1)
    def _final():
        gb = jnp.dot(wbig_ref[...], hh_ref[...],
                     preferred_element_type=jnp.float32)
        g1 = gb[G:2 * G]
        i1, f1, u1, o1 = _gates(g1)
        c2n = f1 * c2_ref[...] + i1 * u1
        h2n = o1 * jnp.tanh(c2n)
        out_ref[...] = jnp.dot(wfc_ref[...], h2n,
                               preferred_element_type=jnp.float32) + bfc_ref[...]


def _pad_gate_rows(w):
    k = w.shape[1]
    return jnp.pad(w.reshape(4, H, k), ((0, 0), (0, HP - H), (0, 0))).reshape(G, k)


def kernel(x, w_ih0, w_hh0, b_ih0, b_hh0, w_ih1, w_hh1, b_ih1, b_hh1,
           w_fc, b_fc):
    B, T = x.shape[0], x.shape[1]
    x2 = x.reshape(B, T, -1)
    I = x2.shape[-1]
    xT = jnp.swapaxes(x2.astype(jnp.bfloat16), 0, 1)

    NB = 2
    Bb = B // NB
    TT = 64
    TC = T // TT

    wi0 = _pad_gate_rows(w_ih0).astype(jnp.bfloat16)
    wh0 = _pad_gate_rows(jnp.pad(w_hh0, ((0, 0), (0, HP - H))))
    b0 = jnp.pad((b_ih0 + b_hh0).reshape(4, H),
                 ((0, 0), (0, HP - H))).reshape(G, 1)
    wi1 = _pad_gate_rows(jnp.pad(w_ih1, ((0, 0), (0, HP - H))))
    wh1 = _pad_gate_rows(jnp.pad(w_hh1, ((0, 0), (0, HP - H))))
    b1 = jnp.pad((b_ih1 + b_hh1).reshape(4, H),
                 ((0, 0), (0, HP - H))).reshape(G, 1)
    b1v = b_ih1 + b_hh1
    c2i = -(jax.nn.sigmoid(b1v[0:H]) * jnp.tanh(b1v[2 * H:3 * H])
            / jax.nn.sigmoid(b1v[H:2 * H]))
    c2i = jnp.pad(c2i, (0, HP - H)).reshape(HP, 1)
    wbig = jnp.concatenate([
        jnp.concatenate([wh0, jnp.zeros_like(wh0)], axis=1),
        jnp.concatenate([wi1, wh1], axis=1),
    ], axis=0)
    bias_col = jnp.concatenate([b0, b1], axis=0)
    wbig = jnp.concatenate(
        [wbig, bias_col,
         jnp.zeros((2 * G, KP - 2 * HP - 1), jnp.float32)], axis=1)
    hh0 = jnp.zeros((KP, 1), jnp.float32).at[2 * HP, 0].set(1.0)
    wfc = jnp.pad(w_fc, ((0, 0), (0, HP - H)))
    O = wfc.shape[0]
    bfc = b_fc.reshape(O, 1)

    full = lambda a: pl.BlockSpec(a.shape, lambda b, t: (0,) * a.ndim)

    outT = pl.pallas_call(
        _lstm_fused_kernel,
        out_shape=jax.ShapeDtypeStruct((O, B), jnp.float32),
        grid=(NB, TC),
        in_specs=[
            pl.BlockSpec((TT, Bb, I), lambda b, t: (t, b, 0)),
            full(wi0), full(wbig), full(hh0), full(c2i),
            full(wfc), full(bfc),
        ],
        out_specs=pl.BlockSpec((O, Bb), lambda b, t: (0, b)),
        scratch_shapes=[
            pltpu.VMEM((KP, Bb), jnp.float32),
            pltpu.VMEM((HP, Bb), jnp.float32),
            pltpu.VMEM((HP, Bb), jnp.float32),
        ],
        compiler_params=pltpu.CompilerParams(
            dimension_semantics=("parallel", "arbitrary"),
            vmem_limit_bytes=56 * 1024 * 1024,
        ),
        name="lstm2_fused",
    )(xT, wi0, wbig, hh0, c2i, wfc, bfc)

    return outT.T.reshape(-1, 16, 9)
